# Initial kernel scaffold; baseline (speedup 1.0000x reference)
import jax
import jax.numpy as jnp
from jax.experimental import pallas as pl


def kernel(user, item, user_tag_embeddings, item_tag_embeddings, user_aspect_bias, item_aspect_bias, global_aspect_bias, user_coeff, item_coeff, global_coeff):
    raise NotImplementedError("write your pallas kernel here")


# speedup vs baseline: 2.0516x; 2.0516x over previous
import jax
import jax.numpy as jnp
from jax import lax
from jax.experimental import pallas as pl
from jax.experimental.pallas import tpu as pltpu
from jax.experimental.pallas import tpu_sc as plsc

B = 16384
T = 26
D = 16
ROW = T * D
N_USER = 100000
N_ITEM = 100000
BW = 32
NBROW_U = N_USER * T // BW
NBROW_I = N_ITEM * T // BW

NC = 2
NS = 16
NW = NC * NS
BPW = B // NW
C = 64
NCHUNK = BPW // C
NG = C // 16


def _body(user_hbm, item_hbm, idx2u_hbm, idx2i_hbm, uemb_hbm, iemb_hbm,
          ub_hbm, ib_hbm, gb_hbm, uc_hbm, ic_hbm, gc_hbm, out_hbm,
          idx_u, idx_i, idx2u, idx2i, urows, irows, ubr, ibr, ucr, icr,
          gbv, gcv, outv, sem):
  wid = lax.axis_index("s") * NC + lax.axis_index("c")
  base = wid * BPW

  pltpu.sync_copy(user_hbm.at[pl.ds(base, BPW)], idx_u)
  pltpu.sync_copy(item_hbm.at[pl.ds(base, BPW)], idx_i)
  pltpu.sync_copy(idx2u_hbm.at[pl.ds(base * 2, BPW * 2)], idx2u)
  pltpu.sync_copy(idx2i_hbm.at[pl.ds(base * 2, BPW * 2)], idx2i)
  pltpu.sync_copy(gb_hbm, gbv)
  pltpu.sync_copy(gc_hbm, gcv)
  gb0 = gbv[pl.ds(0, 16)]
  gb1 = gbv[pl.ds(16, 16)]
  gc0 = gcv[pl.ds(0, 16)]
  gc1 = gcv[pl.ds(16, 16)]

  iota = lax.iota(jnp.int32, 16)

  @pl.loop(0, NCHUNK)
  def _chunk(c):
    iu = idx_u.at[pl.ds(c * C, C)]
    ii = idx_i.at[pl.ds(c * C, C)]
    i2u = idx2u.at[pl.ds(c * 2 * C, 2 * C)]
    i2i = idx2i.at[pl.ds(c * 2 * C, 2 * C)]
    cps = [
        pltpu.async_copy(uemb_hbm.at[iu], urows, sem),
        pltpu.async_copy(iemb_hbm.at[ii], irows, sem),
        pltpu.async_copy(ub_hbm.at[i2u], ubr, sem),
        pltpu.async_copy(ib_hbm.at[i2i], ibr, sem),
        pltpu.async_copy(uc_hbm.at[i2u], ucr, sem),
        pltpu.async_copy(ic_hbm.at[i2i], icr, sem),
    ]
    for cp in cps:
      cp.wait()

    @pl.loop(0, NG)
    def _group(g):
      rvec = iota + g * 16
      uvec = idx_u[pl.ds(c * C + g * 16, 16)]
      ivec = idx_i[pl.ds(c * C + g * 16, 16)]
      au = rvec * (2 * BW) + jnp.bitwise_and(uvec * T, BW - 1)
      ai = rvec * (2 * BW) + jnp.bitwise_and(ivec * T, BW - 1)
      acc = jnp.zeros((16,), jnp.float32)
      for t in range(T):
        s = jnp.zeros((16,), jnp.float32)
        for d in range(D):
          cvec = jnp.full((16,), t * D + d, jnp.int32)
          uu = plsc.load_gather(urows, [rvec, cvec])
          iv = plsc.load_gather(irows, [rvec, cvec])
          s = s + uu * iv
        fu = au + t
        fi = ai + t
        ru = jnp.right_shift(fu, 5)
        cu = jnp.bitwise_and(fu, BW - 1)
        ri = jnp.right_shift(fi, 5)
        ci = jnp.bitwise_and(fi, BW - 1)
        s = s + plsc.load_gather(ubr, [ru, cu])
        s = s + plsc.load_gather(ibr, [ri, ci])
        s = s + (gb0[t] if t < 16 else gb1[t - 16])
        sig = 1.0 / (1.0 + jnp.exp(-s))
        cf = plsc.load_gather(ucr, [ru, cu])
        cf = cf + plsc.load_gather(icr, [ri, ci])
        cf = cf + (gc0[t] if t < 16 else gc1[t - 16])
        acc = acc + sig * cf
      outv[pl.ds(c * C + g * 16, 16)] = acc

  pltpu.sync_copy(outv, out_hbm.at[pl.ds(base, BPW)])


@jax.jit
def _run(user, item, idx2u, idx2i, uemb, iemb, ub, ib, gb, uc, ic, gc):
  mesh = plsc.VectorSubcoreMesh(core_axis_name="c", subcore_axis_name="s")
  f = pl.kernel(
      _body,
      out_type=jax.ShapeDtypeStruct((B,), jnp.float32),
      mesh=mesh,
      scratch_types=[
          pltpu.VMEM((BPW,), jnp.int32),
          pltpu.VMEM((BPW,), jnp.int32),
          pltpu.VMEM((2 * BPW,), jnp.int32),
          pltpu.VMEM((2 * BPW,), jnp.int32),
          pltpu.VMEM((C, ROW), jnp.float32),
          pltpu.VMEM((C, ROW), jnp.float32),
          pltpu.VMEM((2 * C, BW), jnp.float32),
          pltpu.VMEM((2 * C, BW), jnp.float32),
          pltpu.VMEM((2 * C, BW), jnp.float32),
          pltpu.VMEM((2 * C, BW), jnp.float32),
          pltpu.VMEM((32,), jnp.float32),
          pltpu.VMEM((32,), jnp.float32),
          pltpu.VMEM((BPW,), jnp.float32),
          pltpu.SemaphoreType.DMA,
      ],
      compiler_params=pltpu.CompilerParams(use_tc_tiling_on_sc=False,
                                           needs_layout_passes=False),
  )
  return f(user, item, idx2u, idx2i, uemb, iemb, ub, ib, gb, uc, ic, gc)


def _rowpair_indices(idx, nrow):
  r0 = (idx * T) // BW
  r1 = jnp.minimum(r0 + 1, nrow - 1)
  return jnp.stack([r0, r1], axis=1).reshape(-1).astype(jnp.int32)


def kernel(user, item, user_tag_embeddings, item_tag_embeddings,
           user_aspect_bias, item_aspect_bias, global_aspect_bias,
           user_coeff, item_coeff, global_coeff):
  user = user.astype(jnp.int32)
  item = item.astype(jnp.int32)
  uemb = user_tag_embeddings.reshape(-1, ROW)
  iemb = item_tag_embeddings.reshape(-1, ROW)
  ub = user_aspect_bias.reshape(NBROW_U, BW)
  ib = item_aspect_bias.reshape(NBROW_I, BW)
  uc = user_coeff.reshape(NBROW_U, BW)
  ic = item_coeff.reshape(NBROW_I, BW)
  gb = jnp.pad(global_aspect_bias.reshape(T), (0, 32 - T))
  gc = jnp.pad(global_coeff.reshape(T), (0, 32 - T))
  idx2u = _rowpair_indices(user, NBROW_U)
  idx2i = _rowpair_indices(item, NBROW_I)
  return _run(user, item, idx2u, idx2i, uemb, iemb, ub, ib, gb, uc, ic, gc)


# speedup vs baseline: 2.1777x; 1.0614x over previous
import jax
import jax.numpy as jnp
from jax import lax
from jax.experimental import pallas as pl
from jax.experimental.pallas import tpu as pltpu
from jax.experimental.pallas import tpu_sc as plsc

B = 16384
T = 26
D = 16
ROW = T * D
N_USER = 100000
N_ITEM = 100000
BW = 32
NBROW_U = N_USER * T // BW
NBROW_I = N_ITEM * T // BW

NC = 2
NS = 16
NW = NC * NS
BPW = B // NW
C = 64
NCHUNK = BPW // C
NG = C // 16


def _body(user_hbm, item_hbm, idx2u_hbm, idx2i_hbm, uemb_hbm, iemb_hbm,
          ub_hbm, ib_hbm, gb_hbm, uc_hbm, ic_hbm, gc_hbm, out_hbm,
          idx_u, idx_i, idx2u, idx2i, urows, irows, ubr, ibr, ucr, icr,
          gbv, gcv, outv, sem):
  wid = lax.axis_index("s") * NC + lax.axis_index("c")
  base = wid * BPW

  pltpu.sync_copy(user_hbm.at[pl.ds(base, BPW)], idx_u)
  pltpu.sync_copy(item_hbm.at[pl.ds(base, BPW)], idx_i)
  pltpu.sync_copy(idx2u_hbm.at[pl.ds(base * 2, BPW * 2)], idx2u)
  pltpu.sync_copy(idx2i_hbm.at[pl.ds(base * 2, BPW * 2)], idx2i)
  pltpu.sync_copy(gb_hbm, gbv)
  pltpu.sync_copy(gc_hbm, gcv)
  gb0 = gbv[pl.ds(0, 16)]
  gb1 = gbv[pl.ds(16, 16)]
  gc0 = gcv[pl.ds(0, 16)]
  gc1 = gcv[pl.ds(16, 16)]

  iota = lax.iota(jnp.int32, 16)
  rot = [jnp.bitwise_and(iota + d, 15) for d in range(D)]

  @pl.loop(0, NCHUNK)
  def _chunk(c):
    iu = idx_u.at[pl.ds(c * C, C)]
    ii = idx_i.at[pl.ds(c * C, C)]
    i2u = idx2u.at[pl.ds(c * 2 * C, 2 * C)]
    i2i = idx2i.at[pl.ds(c * 2 * C, 2 * C)]
    cps = [
        pltpu.async_copy(uemb_hbm.at[iu], urows, sem),
        pltpu.async_copy(iemb_hbm.at[ii], irows, sem),
        pltpu.async_copy(ub_hbm.at[i2u], ubr, sem),
        pltpu.async_copy(ib_hbm.at[i2i], ibr, sem),
        pltpu.async_copy(uc_hbm.at[i2u], ucr, sem),
        pltpu.async_copy(ic_hbm.at[i2i], icr, sem),
    ]
    for cp in cps:
      cp.wait()

    @pl.loop(0, NG)
    def _group(g):
      rvec = iota + g * 16
      uvec = idx_u[pl.ds(c * C + g * 16, 16)]
      ivec = idx_i[pl.ds(c * C + g * 16, 16)]
      au = rvec * (2 * BW) + jnp.bitwise_and(uvec * T, BW - 1)
      ai = rvec * (2 * BW) + jnp.bitwise_and(ivec * T, BW - 1)
      acc = jnp.zeros((16,), jnp.float32)
      for t in range(T):
        ps = [jnp.zeros((16,), jnp.float32) for _ in range(4)]
        for d in range(D):
          cvec = rot[d] + t * D
          uu = plsc.load_gather(urows, [rvec, cvec])
          iv = plsc.load_gather(irows, [rvec, cvec])
          ps[d % 4] = ps[d % 4] + uu * iv
        s = (ps[0] + ps[1]) + (ps[2] + ps[3])
        fu = au + t
        fi = ai + t
        ru = jnp.right_shift(fu, 5)
        cu = jnp.bitwise_and(fu, BW - 1)
        ri = jnp.right_shift(fi, 5)
        ci = jnp.bitwise_and(fi, BW - 1)
        s = s + plsc.load_gather(ubr, [ru, cu])
        s = s + plsc.load_gather(ibr, [ri, ci])
        s = s + (gb0[t] if t < 16 else gb1[t - 16])
        sig = 1.0 / (1.0 + jnp.exp(-s))
        cf = plsc.load_gather(ucr, [ru, cu])
        cf = cf + plsc.load_gather(icr, [ri, ci])
        cf = cf + (gc0[t] if t < 16 else gc1[t - 16])
        acc = acc + sig * cf
      outv[pl.ds(c * C + g * 16, 16)] = acc

  pltpu.sync_copy(outv, out_hbm.at[pl.ds(base, BPW)])


@jax.jit
def _run(user, item, idx2u, idx2i, uemb, iemb, ub, ib, gb, uc, ic, gc):
  mesh = plsc.VectorSubcoreMesh(core_axis_name="c", subcore_axis_name="s")
  f = pl.kernel(
      _body,
      out_type=jax.ShapeDtypeStruct((B,), jnp.float32),
      mesh=mesh,
      scratch_types=[
          pltpu.VMEM((BPW,), jnp.int32),
          pltpu.VMEM((BPW,), jnp.int32),
          pltpu.VMEM((2 * BPW,), jnp.int32),
          pltpu.VMEM((2 * BPW,), jnp.int32),
          pltpu.VMEM((C, ROW), jnp.float32),
          pltpu.VMEM((C, ROW), jnp.float32),
          pltpu.VMEM((2 * C, BW), jnp.float32),
          pltpu.VMEM((2 * C, BW), jnp.float32),
          pltpu.VMEM((2 * C, BW), jnp.float32),
          pltpu.VMEM((2 * C, BW), jnp.float32),
          pltpu.VMEM((32,), jnp.float32),
          pltpu.VMEM((32,), jnp.float32),
          pltpu.VMEM((BPW,), jnp.float32),
          pltpu.SemaphoreType.DMA,
      ],
      compiler_params=pltpu.CompilerParams(use_tc_tiling_on_sc=False,
                                           needs_layout_passes=False),
  )
  return f(user, item, idx2u, idx2i, uemb, iemb, ub, ib, gb, uc, ic, gc)


def _rowpair_indices(idx, nrow):
  r0 = (idx * T) // BW
  r1 = jnp.minimum(r0 + 1, nrow - 1)
  return jnp.stack([r0, r1], axis=1).reshape(-1).astype(jnp.int32)


def kernel(user, item, user_tag_embeddings, item_tag_embeddings,
           user_aspect_bias, item_aspect_bias, global_aspect_bias,
           user_coeff, item_coeff, global_coeff):
  user = user.astype(jnp.int32)
  item = item.astype(jnp.int32)
  uemb = user_tag_embeddings.reshape(-1, ROW)
  iemb = item_tag_embeddings.reshape(-1, ROW)
  ub = user_aspect_bias.reshape(NBROW_U, BW)
  ib = item_aspect_bias.reshape(NBROW_I, BW)
  uc = user_coeff.reshape(NBROW_U, BW)
  ic = item_coeff.reshape(NBROW_I, BW)
  gb = jnp.pad(global_aspect_bias.reshape(T), (0, 32 - T))
  gc = jnp.pad(global_coeff.reshape(T), (0, 32 - T))
  idx2u = _rowpair_indices(user, NBROW_U)
  idx2i = _rowpair_indices(item, NBROW_I)
  return _run(user, item, idx2u, idx2i, uemb, iemb, ub, ib, gb, uc, ic, gc)
